# baseline (speedup 1.0000x reference)
import jax
import jax.numpy as jnp
import numpy as np
from jax import lax
from jax.experimental import pallas as pl
from jax.experimental.pallas import tpu as pltpu

V = 100000
ROWS = 128
RB = 8
RG = ROWS // RB
CB = 8192
NCB = (V + CB - 1) // CB
CHUNK = 4096
NCH = CB // CHUNK
LANES = 128
NV = CHUNK // LANES

_TAIL_CHUNKS = -(-(V - (NCB - 1) * CB) // CHUNK)

_K0 = 0
_K1 = 42
_KS2 = _K0 ^ _K1 ^ 0x1BD11BDA

_TINY = np.float32(1.1754944e-38)
_NEG_INF = np.float32(float("-inf"))
_BIG_IDX = np.int32(0x7FFFFFFF)


def _threefry2x32_bits(cnt42):
    u32 = jnp.uint32
    rot = lambda v, r: (v << u32(r)) | (v >> u32(32 - r))
    ks = (u32(_K0), u32(_K1), u32(_KS2))
    rotations = ((13, 15, 26, 6), (17, 29, 16, 24))
    inject = ((1, 2), (2, 0), (0, 1), (1, 2), (2, 0))

    x0 = jnp.full(cnt42.shape, ks[0], u32)
    x1 = cnt42
    for g in range(5):
        for r in rotations[g % 2]:
            x0 = x0 + x1
            x1 = rot(x1, r)
            x1 = x1 ^ x0
        a, b = inject[g]
        x0 = x0 + ks[a]
        x1 = x1 + (ks[b] + u32(g + 1))
    return x0 ^ x1


def _gumbel_from_bits(bits):
    float_bits = (bits >> jnp.uint32(9)) | jnp.uint32(0x3F800000)
    f = lax.bitcast_convert_type(float_bits, jnp.float32) - jnp.float32(1.0)
    one = jnp.float32(1.0)
    u = jnp.maximum(_TINY, f * (one - _TINY) + _TINY)
    return -jnp.log(-jnp.log(u))


def _sampler_kernel(logits_ref, temps_ref, out_ref, accm_ref, acci_ref):
    g = pl.program_id(0)
    j = pl.program_id(1)

    t = temps_ref[...]
    tz = t == 0.0
    safe_t = jnp.where(tz, jnp.float32(1.0), t)

    row = g * RB + lax.broadcasted_iota(jnp.int32, (RB, 1), 0)
    rowv42 = (row * V + 42).astype(jnp.uint32)

    lane = lax.broadcasted_iota(jnp.int32, (RB, LANES), 1)
    lane_chunk = lax.broadcasted_iota(jnp.int32, (RB, CHUNK), 1)

    @pl.when(j == 0)
    def _init():
        accm_ref[...] = jnp.full((RB, LANES), _NEG_INF, jnp.float32)
        acci_ref[...] = jnp.zeros((RB, LANES), jnp.int32)

    def body(k, carry):
        accm, acci = carry
        base = j * CB + k * CHUNK
        blk = logits_ref[:, pl.ds(k * CHUNK, CHUNK)]
        col = base + lane_chunk
        cnt42 = rowv42 + col.astype(jnp.uint32)
        gum = _gumbel_from_bits(_threefry2x32_bits(cnt42))
        val = jnp.where(tz, blk, blk / safe_t + gum)
        val = jnp.where(col < V, val, _NEG_INF)

        vs = [val[:, i * LANES:(i + 1) * LANES] for i in range(NV)]
        cs = [base + i * LANES + lane for i in range(NV)]
        while len(vs) > 1:
            nvs, ncs = [], []
            for i in range(0, len(vs), 2):
                lo_v, hi_v = vs[i], vs[i + 1]
                lo_c, hi_c = cs[i], cs[i + 1]
                nvs.append(jnp.maximum(lo_v, hi_v))
                ncs.append(jnp.where(hi_v > lo_v, hi_c, lo_c))
            vs, cs = nvs, ncs
        m, i_ = vs[0], cs[0]

        upd = m > accm
        return jnp.maximum(accm, m), jnp.where(upd, i_, acci)

    tc = jnp.where(j == NCB - 1, _TAIL_CHUNKS, NCH)
    accm, acci = lax.fori_loop(
        0, tc, body, (accm_ref[...], acci_ref[...]))
    accm_ref[...] = accm
    acci_ref[...] = acci

    @pl.when(j == NCB - 1)
    def _emit():
        fm = jnp.max(accm, axis=1, keepdims=True)
        fi = jnp.min(jnp.where(accm == fm, acci, _BIG_IDX),
                     axis=1, keepdims=True)
        out_ref[...] = fi


def kernel(logits, temperatures):
    logits = logits.astype(jnp.float32)
    temps2d = temperatures.reshape(ROWS, 1)
    out = pl.pallas_call(
        _sampler_kernel,
        grid=(RG, NCB),
        in_specs=[
            pl.BlockSpec((RB, CB), lambda g, j: (g, j)),
            pl.BlockSpec((RB, 1), lambda g, j: (g, 0)),
        ],
        out_specs=pl.BlockSpec((RB, 1), lambda g, j: (g, 0)),
        out_shape=jax.ShapeDtypeStruct((ROWS, 1), jnp.int32),
        scratch_shapes=[
            pltpu.VMEM((RB, LANES), jnp.float32),
            pltpu.VMEM((RB, LANES), jnp.int32),
        ],
        compiler_params=pltpu.CompilerParams(
            dimension_semantics=("parallel", "arbitrary"),
        ),
    )(logits, temps2d)
    return out.reshape(ROWS)


# speedup vs baseline: 1.0611x; 1.0611x over previous
import jax
import jax.numpy as jnp
import numpy as np
from jax import lax
from jax.experimental import pallas as pl
from jax.experimental.pallas import tpu as pltpu

V = 100000
ROWS = 128
RB = 8
RG = ROWS // RB
CB = 8192
NCB = (V + CB - 1) // CB
CHUNK = 1024
NCH = CB // CHUNK
LANES = 128
NV = CHUNK // LANES

_TAIL_CHUNKS = -(-(V - (NCB - 1) * CB) // CHUNK)

_K0 = 0
_K1 = 42
_KS2 = _K0 ^ _K1 ^ 0x1BD11BDA

_TINY = np.float32(1.1754944e-38)
_NEG_INF = np.float32(float("-inf"))
_BIG_IDX = np.int32(0x7FFFFFFF)


def _threefry2x32_bits(cnt42):
    u32 = jnp.uint32
    rot = lambda v, r: (v << u32(r)) | (v >> u32(32 - r))
    ks = (u32(_K0), u32(_K1), u32(_KS2))
    rotations = ((13, 15, 26, 6), (17, 29, 16, 24))
    inject = ((1, 2), (2, 0), (0, 1), (1, 2), (2, 0))

    x0 = jnp.full(cnt42.shape, ks[0], u32)
    x1 = cnt42
    for g in range(5):
        for r in rotations[g % 2]:
            x0 = x0 + x1
            x1 = rot(x1, r)
            x1 = x1 ^ x0
        a, b = inject[g]
        x0 = x0 + ks[a]
        x1 = x1 + (ks[b] + u32(g + 1))
    return x0 ^ x1


def _gumbel_from_bits(bits):
    float_bits = (bits >> jnp.uint32(9)) | jnp.uint32(0x3F800000)
    f = lax.bitcast_convert_type(float_bits, jnp.float32) - jnp.float32(1.0)
    one = jnp.float32(1.0)
    u = jnp.maximum(_TINY, f * (one - _TINY) + _TINY)
    return -jnp.log(-jnp.log(u))


def _sampler_kernel(logits_ref, temps_ref, out_ref, accm_ref, acci_ref):
    g = pl.program_id(0)
    j = pl.program_id(1)

    t = temps_ref[...]
    tz = t == 0.0
    safe_t = jnp.where(tz, jnp.float32(1.0), t)

    row = g * RB + lax.broadcasted_iota(jnp.int32, (RB, 1), 0)
    rowv42 = (row * V + 42).astype(jnp.uint32)

    lane = lax.broadcasted_iota(jnp.int32, (RB, LANES), 1)
    lane_chunk = lax.broadcasted_iota(jnp.int32, (RB, CHUNK), 1)

    @pl.when(j == 0)
    def _init():
        accm_ref[...] = jnp.full((RB, LANES), _NEG_INF, jnp.float32)
        acci_ref[...] = jnp.zeros((RB, LANES), jnp.int32)

    def body(k, carry):
        accm, acci = carry
        base = j * CB + k * CHUNK
        blk = logits_ref[:, pl.ds(k * CHUNK, CHUNK)]
        col = base + lane_chunk
        cnt42 = rowv42 + col.astype(jnp.uint32)
        gum = _gumbel_from_bits(_threefry2x32_bits(cnt42))
        val = jnp.where(tz, blk, blk / safe_t + gum)
        val = jnp.where(col < V, val, _NEG_INF)

        vs = [val[:, i * LANES:(i + 1) * LANES] for i in range(NV)]
        cs = [base + i * LANES + lane for i in range(NV)]
        while len(vs) > 1:
            nvs, ncs = [], []
            for i in range(0, len(vs), 2):
                lo_v, hi_v = vs[i], vs[i + 1]
                lo_c, hi_c = cs[i], cs[i + 1]
                nvs.append(jnp.maximum(lo_v, hi_v))
                ncs.append(jnp.where(hi_v > lo_v, hi_c, lo_c))
            vs, cs = nvs, ncs
        m, i_ = vs[0], cs[0]

        upd = m > accm
        return jnp.maximum(accm, m), jnp.where(upd, i_, acci)

    tc = jnp.where(j == NCB - 1, _TAIL_CHUNKS, NCH)
    accm, acci = lax.fori_loop(
        0, tc, body, (accm_ref[...], acci_ref[...]))
    accm_ref[...] = accm
    acci_ref[...] = acci

    @pl.when(j == NCB - 1)
    def _emit():
        fm = jnp.max(accm, axis=1, keepdims=True)
        fi = jnp.min(jnp.where(accm == fm, acci, _BIG_IDX),
                     axis=1, keepdims=True)
        out_ref[...] = fi


def kernel(logits, temperatures):
    logits = logits.astype(jnp.float32)
    temps2d = temperatures.reshape(ROWS, 1)
    out = pl.pallas_call(
        _sampler_kernel,
        grid=(RG, NCB),
        in_specs=[
            pl.BlockSpec((RB, CB), lambda g, j: (g, j)),
            pl.BlockSpec((RB, 1), lambda g, j: (g, 0)),
        ],
        out_specs=pl.BlockSpec((RB, 1), lambda g, j: (g, 0)),
        out_shape=jax.ShapeDtypeStruct((ROWS, 1), jnp.int32),
        scratch_shapes=[
            pltpu.VMEM((RB, LANES), jnp.float32),
            pltpu.VMEM((RB, LANES), jnp.int32),
        ],
        compiler_params=pltpu.CompilerParams(
            dimension_semantics=("parallel", "arbitrary"),
        ),
    )(logits, temps2d)
    return out.reshape(ROWS)


# speedup vs baseline: 1.0727x; 1.0109x over previous
import jax
import jax.numpy as jnp
import numpy as np
from jax import lax
from jax.experimental import pallas as pl
from jax.experimental.pallas import tpu as pltpu

V = 100000
ROWS = 128
RB = 8
RG = ROWS // RB
CB = 8192
NCB = (V + CB - 1) // CB
CHUNK = 2048
NCH = CB // CHUNK
LANES = 128
NV = CHUNK // LANES

_TAIL_CHUNKS = -(-(V - (NCB - 1) * CB) // CHUNK)

_K0 = 0
_K1 = 42
_KS2 = _K0 ^ _K1 ^ 0x1BD11BDA

_TINY = np.float32(1.1754944e-38)
_NEG_INF = np.float32(float("-inf"))
_BIG_IDX = np.int32(0x7FFFFFFF)


def _threefry2x32_bits(cnt42):
    u32 = jnp.uint32
    rot = lambda v, r: (v << u32(r)) | (v >> u32(32 - r))
    ks = (u32(_K0), u32(_K1), u32(_KS2))
    rotations = ((13, 15, 26, 6), (17, 29, 16, 24))
    inject = ((1, 2), (2, 0), (0, 1), (1, 2), (2, 0))

    x0 = jnp.full(cnt42.shape, ks[0], u32)
    x1 = cnt42
    for g in range(5):
        for r in rotations[g % 2]:
            x0 = x0 + x1
            x1 = rot(x1, r)
            x1 = x1 ^ x0
        a, b = inject[g]
        x0 = x0 + ks[a]
        x1 = x1 + (ks[b] + u32(g + 1))
    return x0 ^ x1


def _gumbel_from_bits(bits):
    float_bits = (bits >> jnp.uint32(9)) | jnp.uint32(0x3F800000)
    f = lax.bitcast_convert_type(float_bits, jnp.float32) - jnp.float32(1.0)
    one = jnp.float32(1.0)
    u = jnp.maximum(_TINY, f * (one - _TINY) + _TINY)
    return -jnp.log(-jnp.log(u))


def _sampler_kernel(logits_ref, temps_ref, out_ref, accm_ref, acci_ref):
    g = pl.program_id(0)
    j = pl.program_id(1)

    t = temps_ref[...]
    tz = t == 0.0
    safe_t = jnp.where(tz, jnp.float32(1.0), t)

    row = g * RB + lax.broadcasted_iota(jnp.int32, (RB, 1), 0)
    rowv42 = (row * V + 42).astype(jnp.uint32)

    lane = lax.broadcasted_iota(jnp.int32, (RB, LANES), 1)
    lane_chunk = lax.broadcasted_iota(jnp.int32, (RB, CHUNK), 1)

    @pl.when(j == 0)
    def _init():
        accm_ref[...] = jnp.full((RB, LANES), _NEG_INF, jnp.float32)
        acci_ref[...] = jnp.zeros((RB, LANES), jnp.int32)

    def body(k, carry):
        accm, acci = carry
        base = j * CB + k * CHUNK
        blk = logits_ref[:, pl.ds(k * CHUNK, CHUNK)]
        col = base + lane_chunk
        cnt42 = rowv42 + col.astype(jnp.uint32)
        gum = _gumbel_from_bits(_threefry2x32_bits(cnt42))
        val = jnp.where(tz, blk, blk / safe_t + gum)
        val = jnp.where(col < V, val, _NEG_INF)

        vs = [val[:, i * LANES:(i + 1) * LANES] for i in range(NV)]
        cs = [base + i * LANES + lane for i in range(NV)]
        while len(vs) > 1:
            nvs, ncs = [], []
            for i in range(0, len(vs), 2):
                lo_v, hi_v = vs[i], vs[i + 1]
                lo_c, hi_c = cs[i], cs[i + 1]
                nvs.append(jnp.maximum(lo_v, hi_v))
                ncs.append(jnp.where(hi_v > lo_v, hi_c, lo_c))
            vs, cs = nvs, ncs
        m, i_ = vs[0], cs[0]

        upd = m > accm
        return jnp.maximum(accm, m), jnp.where(upd, i_, acci)

    accm, acci = accm_ref[...], acci_ref[...]
    for k in range(NCH):
        accm, acci = body(k, (accm, acci))
    accm_ref[...] = accm
    acci_ref[...] = acci

    @pl.when(j == NCB - 1)
    def _emit():
        fm = jnp.max(accm, axis=1, keepdims=True)
        fi = jnp.min(jnp.where(accm == fm, acci, _BIG_IDX),
                     axis=1, keepdims=True)
        out_ref[...] = fi


def kernel(logits, temperatures):
    logits = logits.astype(jnp.float32)
    temps2d = temperatures.reshape(ROWS, 1)
    out = pl.pallas_call(
        _sampler_kernel,
        grid=(RG, NCB),
        in_specs=[
            pl.BlockSpec((RB, CB), lambda g, j: (g, j)),
            pl.BlockSpec((RB, 1), lambda g, j: (g, 0)),
        ],
        out_specs=pl.BlockSpec((RB, 1), lambda g, j: (g, 0)),
        out_shape=jax.ShapeDtypeStruct((ROWS, 1), jnp.int32),
        scratch_shapes=[
            pltpu.VMEM((RB, LANES), jnp.float32),
            pltpu.VMEM((RB, LANES), jnp.int32),
        ],
        compiler_params=pltpu.CompilerParams(
            dimension_semantics=("parallel", "arbitrary"),
        ),
    )(logits, temps2d)
    return out.reshape(ROWS)


# speedup vs baseline: 1.1291x; 1.0526x over previous
import jax
import jax.numpy as jnp
import numpy as np
from jax import lax
from jax.experimental import pallas as pl
from jax.experimental.pallas import tpu as pltpu

V = 100000
ROWS = 128
RB = 8
RG = ROWS // RB
CB = 16384
NCB = (V + CB - 1) // CB
CHUNK = 2048
NCH = CB // CHUNK
_TAIL_CHUNKS = -(-(V - (NCB - 1) * CB) // CHUNK)

_K0 = np.uint32(0)
_K1 = np.uint32(42)
_KS2 = np.uint32(_K0 ^ _K1 ^ np.uint32(0x1BD11BDA))

_TINY = np.float32(1.1754944e-38)
_NEG_INF = np.float32(float("-inf"))
_BIG_IDX = np.int32(0x7FFFFFFF)


def _threefry2x32_bits(cnt42):
    u32 = jnp.uint32
    rot = lambda v, r: (v << u32(r)) | (v >> u32(32 - r))
    ks = (_K0, _K1, _KS2)
    rotations = ((13, 15, 26, 6), (17, 29, 16, 24))
    inject = ((1, 2), (2, 0), (0, 1), (1, 2), (2, 0))

    x0 = cnt42
    x1 = cnt42
    first = True
    for g in range(5):
        for r in rotations[g % 2]:
            if first:
                first = False
            else:
                x0 = x0 + x1
            x1 = rot(x1, r)
            x1 = x1 ^ x0
        a, b = inject[g]
        x0 = x0 + ks[a]
        x1 = x1 + np.uint32(ks[b] + np.uint32(g + 1))
    return x0 ^ x1


def _gumbel_from_bits(bits):
    float_bits = (bits >> jnp.uint32(9)) | jnp.uint32(0x3F800000)
    f = lax.bitcast_convert_type(float_bits, jnp.float32) - jnp.float32(1.0)
    u = f + _TINY
    return -jnp.log(-jnp.log(u))


def _sampler_kernel(logits_ref, temps_ref, out_ref, accm_ref, acci_ref):
    g = pl.program_id(0)
    j = pl.program_id(1)

    t = temps_ref[...]
    tz = t == 0.0
    safe_t = jnp.where(tz, jnp.float32(1.0), t)

    row = g * RB + lax.broadcasted_iota(jnp.int32, (RB, 1), 0)
    rowv42 = (row * V + 42).astype(jnp.uint32)

    lane_chunk = lax.broadcasted_iota(jnp.int32, (RB, CHUNK), 1)

    @pl.when(j == 0)
    def _init():
        accm_ref[...] = jnp.full((RB, CHUNK), _NEG_INF, jnp.float32)
        acci_ref[...] = jnp.zeros((RB, CHUNK), jnp.int32)

    def chunk_val(k):
        base = j * CB + k * CHUNK
        blk = logits_ref[:, k * CHUNK:(k + 1) * CHUNK]
        col = base + lane_chunk
        cnt42 = rowv42 + col.astype(jnp.uint32)
        gum = _gumbel_from_bits(_threefry2x32_bits(cnt42))
        return col, jnp.where(tz, blk, blk / safe_t + gum)

    def update(accm, acci, col, val):
        upd = val > accm
        return jnp.maximum(accm, val), jnp.where(upd, col, acci)

    @pl.when(j < NCB - 1)
    def _main():
        accm, acci = accm_ref[...], acci_ref[...]
        for k in range(NCH):
            col, val = chunk_val(k)
            accm, acci = update(accm, acci, col, val)
        accm_ref[...] = accm
        acci_ref[...] = acci

    @pl.when(j == NCB - 1)
    def _tail():
        accm, acci = accm_ref[...], acci_ref[...]
        for k in range(_TAIL_CHUNKS):
            col, val = chunk_val(k)
            val = jnp.where(col < V, val, _NEG_INF)
            accm, acci = update(accm, acci, col, val)
        fm = jnp.max(accm, axis=1, keepdims=True)
        fi = jnp.min(jnp.where(accm == fm, acci, _BIG_IDX),
                     axis=1, keepdims=True)
        out_ref[...] = fi


def kernel(logits, temperatures):
    logits = logits.astype(jnp.float32)
    temps2d = temperatures.reshape(ROWS, 1)
    out = pl.pallas_call(
        _sampler_kernel,
        grid=(RG, NCB),
        in_specs=[
            pl.BlockSpec((RB, CB), lambda g, j: (g, j)),
            pl.BlockSpec((RB, 1), lambda g, j: (g, 0)),
        ],
        out_specs=pl.BlockSpec((RB, 1), lambda g, j: (g, 0)),
        out_shape=jax.ShapeDtypeStruct((ROWS, 1), jnp.int32),
        scratch_shapes=[
            pltpu.VMEM((RB, CHUNK), jnp.float32),
            pltpu.VMEM((RB, CHUNK), jnp.int32),
        ],
        compiler_params=pltpu.CompilerParams(
            dimension_semantics=("parallel", "arbitrary"),
        ),
    )(logits, temps2d)
    return out.reshape(ROWS)


# speedup vs baseline: 1.1695x; 1.0357x over previous
import jax
import jax.numpy as jnp
import numpy as np
from jax import lax
from jax.experimental import pallas as pl
from jax.experimental.pallas import tpu as pltpu

V = 100000
ROWS = 128
RB = 8
RG = ROWS // RB
CB = 16384
NCB = (V + CB - 1) // CB
CHUNK = 1024
NCH = CB // CHUNK
_TAIL_CHUNKS = -(-(V - (NCB - 1) * CB) // CHUNK)

_K0 = np.uint32(0)
_K1 = np.uint32(42)
_KS2 = np.uint32(_K0 ^ _K1 ^ np.uint32(0x1BD11BDA))

_TINY = np.float32(1.1754944e-38)
_NEG_INF = np.float32(float("-inf"))
_BIG_IDX = np.int32(0x7FFFFFFF)


def _threefry2x32_bits(cnt42):
    u32 = jnp.uint32
    rot = lambda v, r: (v << u32(r)) | (v >> u32(32 - r))
    ks = (_K0, _K1, _KS2)
    rotations = ((13, 15, 26, 6), (17, 29, 16, 24))
    inject = ((1, 2), (2, 0), (0, 1), (1, 2), (2, 0))

    x0 = cnt42
    x1 = cnt42
    first = True
    for g in range(5):
        for r in rotations[g % 2]:
            if first:
                first = False
            else:
                x0 = x0 + x1
            x1 = rot(x1, r)
            x1 = x1 ^ x0
        a, b = inject[g]
        x0 = x0 + ks[a]
        x1 = x1 + np.uint32(ks[b] + np.uint32(g + 1))
    return x0 ^ x1


def _gumbel_from_bits(bits):
    float_bits = (bits >> jnp.uint32(9)) | jnp.uint32(0x3F800000)
    f = lax.bitcast_convert_type(float_bits, jnp.float32) - jnp.float32(1.0)
    u = f + _TINY
    return -jnp.log(-jnp.log(u))


def _sampler_kernel(logits_ref, temps_ref, out_ref, accm_ref, acci_ref):
    g = pl.program_id(0)
    j = pl.program_id(1)

    t = temps_ref[...]
    tz = t == 0.0
    safe_t = jnp.where(tz, jnp.float32(1.0), t)

    row = g * RB + lax.broadcasted_iota(jnp.int32, (RB, 1), 0)
    rowv42 = (row * V + 42).astype(jnp.uint32)

    lane_chunk = lax.broadcasted_iota(jnp.int32, (RB, CHUNK), 1)

    @pl.when(j == 0)
    def _init():
        accm_ref[...] = jnp.full((RB, CHUNK), _NEG_INF, jnp.float32)
        acci_ref[...] = jnp.zeros((RB, CHUNK), jnp.int32)

    def chunk_val(k):
        base = j * CB + k * CHUNK
        blk = logits_ref[:, k * CHUNK:(k + 1) * CHUNK]
        col = base + lane_chunk
        cnt42 = rowv42 + col.astype(jnp.uint32)
        gum = _gumbel_from_bits(_threefry2x32_bits(cnt42))
        return col, jnp.where(tz, blk, blk / safe_t + gum)

    def update(accm, acci, col, val):
        upd = val > accm
        return jnp.maximum(accm, val), jnp.where(upd, col, acci)

    @pl.when(j < NCB - 1)
    def _main():
        accm, acci = accm_ref[...], acci_ref[...]
        for k in range(NCH):
            col, val = chunk_val(k)
            accm, acci = update(accm, acci, col, val)
        accm_ref[...] = accm
        acci_ref[...] = acci

    @pl.when(j == NCB - 1)
    def _tail():
        accm, acci = accm_ref[...], acci_ref[...]
        for k in range(_TAIL_CHUNKS):
            col, val = chunk_val(k)
            val = jnp.where(col < V, val, _NEG_INF)
            accm, acci = update(accm, acci, col, val)
        fm = jnp.max(accm, axis=1, keepdims=True)
        fi = jnp.min(jnp.where(accm == fm, acci, _BIG_IDX),
                     axis=1, keepdims=True)
        out_ref[...] = fi


def kernel(logits, temperatures):
    logits = logits.astype(jnp.float32)
    temps2d = temperatures.reshape(ROWS, 1)
    out = pl.pallas_call(
        _sampler_kernel,
        grid=(RG, NCB),
        in_specs=[
            pl.BlockSpec((RB, CB), lambda g, j: (g, j)),
            pl.BlockSpec((RB, 1), lambda g, j: (g, 0)),
        ],
        out_specs=pl.BlockSpec((RB, 1), lambda g, j: (g, 0)),
        out_shape=jax.ShapeDtypeStruct((ROWS, 1), jnp.int32),
        scratch_shapes=[
            pltpu.VMEM((RB, CHUNK), jnp.float32),
            pltpu.VMEM((RB, CHUNK), jnp.int32),
        ],
        compiler_params=pltpu.CompilerParams(
            dimension_semantics=("parallel", "arbitrary"),
        ),
    )(logits, temps2d)
    return out.reshape(ROWS)
